# baseline (speedup 1.0000x reference)
import dataclasses
import functools

import jax
import jax.numpy as jnp
from jax import lax
from jax.experimental import pallas as pl
from jax.experimental.pallas import tpu as pltpu
from jax.experimental.pallas import tpu_sc as plsc

NC = 2
NS = 16
NW = NC * NS
LANES = 16


def _make_kernel(n_nodes, feat, n_edges, chunk):
    per_tile = n_edges // NW
    n_chunks = per_tile // chunk
    assert per_tile % chunk == 0 and chunk % LANES == 0 and chunk % 8 == 0
    assert n_chunks % 2 == 1
    mesh = plsc.VectorSubcoreMesh(core_axis_name="c", subcore_axis_name="s")
    cp = pltpu.CompilerParams()
    for field, val in (("needs_layout_passes", False),
                       ("use_tc_tiling_on_sc", False),
                       ("disable_bounds_checks", True),
                       ("disable_semaphore_checks", True)):
        if field in pltpu.CompilerParams.__dataclass_fields__:
            cp = dataclasses.replace(cp, **{field: val})

    @functools.partial(
        pl.kernel,
        mesh=mesh,
        compiler_params=cp,
        out_type=[
            jax.ShapeDtypeStruct((n_edges,), jnp.float32),
            jax.ShapeDtypeStruct((NC, n_nodes, feat // 2), jnp.int32),
        ],
        scratch_types=[
            pltpu.VMEM((per_tile,), jnp.int32),
            pltpu.VMEM((per_tile,), jnp.int32),
            pltpu.VMEM((n_nodes // NS // 5, feat), jnp.float32),
            pltpu.VMEM((n_nodes // NS // 5, feat), jnp.float32),
            pltpu.VMEM((n_nodes // NS // 5, feat // 2), jnp.int32),
            pltpu.VMEM((n_nodes // NS // 5, feat // 2), jnp.int32),
            pltpu.VMEM((chunk, feat // 2), jnp.int32),
            pltpu.VMEM((chunk, feat // 2), jnp.int32),
            pltpu.VMEM((chunk, feat // 2), jnp.int32),
            pltpu.VMEM((chunk, feat // 2), jnp.int32),
            pltpu.VMEM((per_tile,), jnp.float32),
            pltpu.VMEM((LANES * LANES,), jnp.float32),
            pltpu.VMEM((LANES * LANES,), jnp.float32),
            pltpu.SemaphoreType.DMA,
            pltpu.SemaphoreType.DMA,
            pltpu.SemaphoreType.DMA,
            pltpu.SemaphoreType.DMA,
        ],
    )
    def k(x_hbm, ei_hbm, out_hbm, pk_hbm,
          idx_s, idx_d, cvi0, cvi1, cvo0, cvo1, rs0, rd0, rs1, rd1, out_v,
          part_a, part_b, sem_s0, sem_d0, sem_s1, sem_d1):
        cid = lax.axis_index("c")
        sid = lax.axis_index("s")
        wid = sid * NC + cid
        tile_base = wid * per_tile
        col_idx = lax.iota(jnp.int32, LANES) * LANES

        pltpu.sync_copy(ei_hbm.at[0, pl.ds(tile_base, per_tile)], idx_s)
        pltpu.sync_copy(ei_hbm.at[1, pl.ds(tile_base, per_tile)], idx_d)

        pk = pk_hbm.at[cid]
        cv_rows = n_nodes // NS // 5
        cv_base = sid * (n_nodes // NS)

        def conv(ci, co):
            @pl.loop(0, cv_rows)
            def _(r):
                for c in range(feat // (2 * LANES)):
                    a = ci[r, pl.ds(c * 2 * LANES, LANES)]
                    b = ci[r, pl.ds(c * 2 * LANES + LANES, LANES)]
                    pv = plsc.pack(a, b, format=plsc.PackFormat.INTERLEAVED)
                    co[r, pl.ds(c * LANES, LANES)] = plsc.bitcast(
                        pv, jnp.int32)

        def cv_in_start(t, ci, sem):
            return pltpu.async_copy(
                x_hbm.at[pl.ds(cv_base + t * cv_rows, cv_rows)], ci, sem)

        def cv_out_start(t, co, sem):
            return pltpu.async_copy(
                co, pk.at[pl.ds(cv_base + t * cv_rows, cv_rows)], sem)

        h_in0 = cv_in_start(0, cvi0, sem_s0)
        h_in1 = cv_in_start(1, cvi1, sem_d0)
        h_in0.wait()
        conv(cvi0, cvo0)
        h_out0 = cv_out_start(0, cvo0, sem_s1)
        h_in2 = cv_in_start(2, cvi0, sem_s0)
        h_in1.wait()
        conv(cvi1, cvo1)
        h_out1 = cv_out_start(1, cvo1, sem_d1)
        h_in3 = cv_in_start(3, cvi1, sem_d0)
        h_in2.wait()
        h_out0.wait()
        conv(cvi0, cvo0)
        h_out2 = cv_out_start(2, cvo0, sem_s1)
        h_in4 = cv_in_start(4, cvi0, sem_s0)
        h_in3.wait()
        h_out1.wait()
        conv(cvi1, cvo1)
        h_out3 = cv_out_start(3, cvo1, sem_d1)
        h_in4.wait()
        h_out2.wait()
        conv(cvi0, cvo0)
        h_out4 = cv_out_start(4, cvo0, sem_s1)
        h_out3.wait()
        h_out4.wait()

        plsc.subcore_barrier()

        def issue(j, rs, rd, sem_s, sem_d):
            pltpu.async_copy(
                pk.at[idx_s.at[pl.ds(j * chunk, chunk)]], rs, sem_s)
            pltpu.async_copy(
                pk.at[idx_d.at[pl.ds(j * chunk, chunk)]], rd, sem_d)

        def wait(rs, rd, sem_s, sem_d):
            pltpu.make_async_copy(pk.at[pl.ds(0, chunk)], rs, sem_s).wait()
            pltpu.make_async_copy(pk.at[pl.ds(0, chunk)], rd, sem_d).wait()

        def edge_acc(rows_s, rows_d, e):
            m = [plsc.bitcast(rows_s[e, pl.ds(c * LANES, LANES)],
                              jnp.bfloat16)
                 * plsc.bitcast(rows_d[e, pl.ds(c * LANES, LANES)],
                                jnp.bfloat16)
                 for c in range(feat // (2 * LANES))]
            while len(m) > 1:
                m = [m[i] + m[i + 1] for i in range(0, len(m), 2)]
            lo, hi = plsc.unpack(m[0], format=plsc.PackFormat.INTERLEAVED)
            return lo + hi

        def do_group(base, g, rows_s, rows_d, part):
            e0 = g * LANES
            accs = [edge_acc(rows_s, rows_d, e0 + el) for el in range(LANES)]
            for el in range(LANES):
                plsc.store_scatter(part, [col_idx + el], accs[el])
            rows = [part[pl.ds(i * LANES, LANES)] for i in range(LANES)]
            while len(rows) > 1:
                rows = [rows[i] + rows[i + 1] for i in range(0, len(rows), 2)]
            out_v[pl.ds(base + e0, LANES)] = rows[0]

        def big_group(base, G, rows_s, rows_d, part):
            eb = G * 2 * LANES
            for half in range(2):
                pws = []
                for p in range(half * 8, half * 8 + 8):
                    a0 = edge_acc(rows_s, rows_d, eb + p)
                    a1 = edge_acc(rows_s, rows_d, eb + p + LANES)
                    pws.append((p, plsc.bitcast(
                        plsc.pack(a0, a1, format=plsc.PackFormat.INTERLEAVED),
                        jnp.float32)))
                for p, w in pws:
                    plsc.store_scatter(part, [col_idx + p], w)
            rows = [plsc.bitcast(part[pl.ds(i * LANES, LANES)], jnp.bfloat16)
                    for i in range(LANES)]
            while len(rows) > 1:
                rows = [rows[i] + rows[i + 1] for i in range(0, len(rows), 2)]
            lo, hi = plsc.unpack(rows[0], format=plsc.PackFormat.INTERLEAVED)
            out_v[pl.ds(base + eb, LANES)] = lo
            out_v[pl.ds(base + eb + LANES, LANES)] = hi

        def compute(j, rows_s, rows_d):
            base = j * chunk
            big_group(base, 0, rows_s, rows_d, part_a)
            big_group(base, 1, rows_s, rows_d, part_b)
            do_group(base, 4, rows_s, rows_d, part_a)

        issue(0, rs0, rd0, sem_s0, sem_d0)

        @pl.loop(0, (n_chunks - 1) // 2)
        def _(jj):
            j = jj * 2
            issue(j + 1, rs1, rd1, sem_s1, sem_d1)
            wait(rs0, rd0, sem_s0, sem_d0)
            compute(j, rs0, rd0)
            issue(j + 2, rs0, rd0, sem_s0, sem_d0)
            wait(rs1, rd1, sem_s1, sem_d1)
            compute(j + 1, rs1, rd1)

        wait(rs0, rd0, sem_s0, sem_d0)
        compute(n_chunks - 1, rs0, rd0)

        pltpu.sync_copy(out_v, out_hbm.at[pl.ds(tile_base, per_tile)])

    return k


def kernel(x, edge_index):
    n_nodes, feat = x.shape
    n_edges = edge_index.shape[1]
    k = _make_kernel(n_nodes, feat, n_edges, chunk=80)
    out, _ = k(x, edge_index.astype(jnp.int32))
    return out


# speedup vs baseline: 1.5146x; 1.5146x over previous
import dataclasses
import functools

import jax
import jax.numpy as jnp
from jax import lax
from jax.experimental import pallas as pl
from jax.experimental.pallas import tpu as pltpu
from jax.experimental.pallas import tpu_sc as plsc

NC = 2
NS = 16
NW = NC * NS
LANES = 16


def _make_kernel(n_nodes, feat, n_edges, chunk):
    per_tile = n_edges // NW
    n_chunks = per_tile // chunk
    assert per_tile % chunk == 0 and chunk % LANES == 0 and chunk % 8 == 0
    assert n_chunks % 2 == 1
    mesh = plsc.VectorSubcoreMesh(core_axis_name="c", subcore_axis_name="s")
    cp = pltpu.CompilerParams()
    for field, val in (("needs_layout_passes", False),
                       ("use_tc_tiling_on_sc", False),
                       ("disable_bounds_checks", True),
                       ("disable_semaphore_checks", True)):
        if field in pltpu.CompilerParams.__dataclass_fields__:
            cp = dataclasses.replace(cp, **{field: val})

    @functools.partial(
        pl.kernel,
        mesh=mesh,
        compiler_params=cp,
        out_type=[
            jax.ShapeDtypeStruct((n_edges,), jnp.float32),
            jax.ShapeDtypeStruct((NC, n_nodes, feat // 2), jnp.int32),
        ],
        scratch_types=[
            pltpu.VMEM((per_tile,), jnp.int32),
            pltpu.VMEM((per_tile,), jnp.int32),
            pltpu.VMEM((n_nodes // NS // 5, feat), jnp.float32),
            pltpu.VMEM((n_nodes // NS // 5, feat), jnp.float32),
            pltpu.VMEM((n_nodes // NS // 5, feat // 2), jnp.int32),
            pltpu.VMEM((n_nodes // NS // 5, feat // 2), jnp.int32),
            pltpu.VMEM((chunk, feat // 2), jnp.int32),
            pltpu.VMEM((chunk, feat // 2), jnp.int32),
            pltpu.VMEM((chunk, feat // 2), jnp.int32),
            pltpu.VMEM((chunk, feat // 2), jnp.int32),
            pltpu.VMEM((per_tile,), jnp.float32),
            pltpu.VMEM((LANES * LANES,), jnp.float32),
            pltpu.VMEM((LANES * LANES,), jnp.float32),
            pltpu.SemaphoreType.DMA,
            pltpu.SemaphoreType.DMA,
            pltpu.SemaphoreType.DMA,
            pltpu.SemaphoreType.DMA,
        ],
    )
    def k(x_hbm, ei_hbm, out_hbm, pk_hbm,
          idx_s, idx_d, cvi0, cvi1, cvo0, cvo1, rs0, rd0, rs1, rd1, out_v,
          part_a, part_b, sem_s0, sem_d0, sem_s1, sem_d1):
        cid = lax.axis_index("c")
        sid = lax.axis_index("s")
        wid = sid * NC + cid
        tile_base = wid * per_tile
        col_idx = lax.iota(jnp.int32, LANES) * LANES

        pltpu.sync_copy(ei_hbm.at[0, pl.ds(tile_base, per_tile)], idx_s)
        pltpu.sync_copy(ei_hbm.at[1, pl.ds(tile_base, per_tile)], idx_d)

        pk = pk_hbm.at[cid]
        cv_rows = n_nodes // NS // 5
        cv_base = sid * (n_nodes // NS)

        def conv(ci, co):
            @pl.loop(0, cv_rows)
            def _(r):
                for c in range(feat // (2 * LANES)):
                    a = ci[r, pl.ds(c * 2 * LANES, LANES)]
                    b = ci[r, pl.ds(c * 2 * LANES + LANES, LANES)]
                    pv = plsc.pack(a, b, format=plsc.PackFormat.INTERLEAVED)
                    co[r, pl.ds(c * LANES, LANES)] = plsc.bitcast(
                        pv, jnp.int32)

        def cv_in_start(t, ci, sem):
            return pltpu.async_copy(
                x_hbm.at[pl.ds(cv_base + t * cv_rows, cv_rows)], ci, sem)

        def cv_out_start(t, co, sem):
            return pltpu.async_copy(
                co, pk.at[pl.ds(cv_base + t * cv_rows, cv_rows)], sem)

        h_in0 = cv_in_start(0, cvi0, sem_s0)
        h_in1 = cv_in_start(1, cvi1, sem_d0)
        h_in0.wait()
        conv(cvi0, cvo0)
        h_out0 = cv_out_start(0, cvo0, sem_s1)
        h_in2 = cv_in_start(2, cvi0, sem_s0)
        h_in1.wait()
        conv(cvi1, cvo1)
        h_out1 = cv_out_start(1, cvo1, sem_d1)
        h_in3 = cv_in_start(3, cvi1, sem_d0)
        h_in2.wait()
        h_out0.wait()
        conv(cvi0, cvo0)
        h_out2 = cv_out_start(2, cvo0, sem_s1)
        h_in4 = cv_in_start(4, cvi0, sem_s0)
        h_in3.wait()
        h_out1.wait()
        conv(cvi1, cvo1)
        h_out3 = cv_out_start(3, cvo1, sem_d1)
        h_in4.wait()
        h_out2.wait()
        conv(cvi0, cvo0)
        h_out4 = cv_out_start(4, cvo0, sem_s1)
        h_out3.wait()
        h_out4.wait()

        plsc.subcore_barrier()

        def issue(j, rs, rd, sem_s, sem_d):
            pltpu.async_copy(
                pk.at[idx_s.at[pl.ds(j * chunk, chunk)]], rs, sem_s)
            pltpu.async_copy(
                pk.at[idx_d.at[pl.ds(j * chunk, chunk)]], rd, sem_d)

        def wait(rs, rd, sem_s, sem_d):
            pltpu.make_async_copy(pk.at[pl.ds(0, chunk)], rs, sem_s).wait()
            pltpu.make_async_copy(pk.at[pl.ds(0, chunk)], rd, sem_d).wait()

        def do_group(base, g, rows_s, rows_d, part):
            e0 = g * LANES
            accs = []
            for el in range(LANES):
                e = e0 + el
                m = [plsc.bitcast(rows_s[e, pl.ds(c * LANES, LANES)],
                                  jnp.bfloat16)
                     * plsc.bitcast(rows_d[e, pl.ds(c * LANES, LANES)],
                                    jnp.bfloat16)
                     for c in range(feat // (2 * LANES))]
                while len(m) > 1:
                    m = [m[i] + m[i + 1] for i in range(0, len(m), 2)]
                lo, hi = plsc.unpack(m[0], format=plsc.PackFormat.INTERLEAVED)
                accs.append(lo + hi)
            for el in range(LANES):
                plsc.store_scatter(part, [col_idx + el], accs[el])
            rows = [part[pl.ds(i * LANES, LANES)] for i in range(LANES)]
            while len(rows) > 1:
                rows = [rows[i] + rows[i + 1] for i in range(0, len(rows), 2)]
            out_v[pl.ds(base + e0, LANES)] = rows[0]

        def compute(j, rows_s, rows_d):
            base = j * chunk
            n_groups = chunk // LANES

            @pl.loop(0, n_groups // 2)
            def _(i):
                do_group(base, i * 2, rows_s, rows_d, part_a)
                do_group(base, i * 2 + 1, rows_s, rows_d, part_b)

            if n_groups % 2:
                do_group(base, n_groups - 1, rows_s, rows_d, part_a)

        issue(0, rs0, rd0, sem_s0, sem_d0)

        @pl.loop(0, (n_chunks - 1) // 2)
        def _(jj):
            j = jj * 2
            issue(j + 1, rs1, rd1, sem_s1, sem_d1)
            wait(rs0, rd0, sem_s0, sem_d0)
            compute(j, rs0, rd0)
            issue(j + 2, rs0, rd0, sem_s0, sem_d0)
            wait(rs1, rd1, sem_s1, sem_d1)
            compute(j + 1, rs1, rd1)

        wait(rs0, rd0, sem_s0, sem_d0)
        compute(n_chunks - 1, rs0, rd0)

        pltpu.sync_copy(out_v, out_hbm.at[pl.ds(tile_base, per_tile)])

    return k


def kernel(x, edge_index):
    n_nodes, feat = x.shape
    n_edges = edge_index.shape[1]
    k = _make_kernel(n_nodes, feat, n_edges, chunk=80)
    out, _ = k(x, edge_index.astype(jnp.int32))
    return out
